# baseline (speedup 1.0000x reference)
import dataclasses
import functools

import jax
import jax.numpy as jnp
import numpy as np
from jax import lax
from jax.experimental import pallas as pl
from jax.experimental.pallas import tpu as pltpu
from jax.experimental.pallas import tpu_sc as plsc

N = 10000
NP = 10240
E = 320000
D = 128
DH = 64
NC = 2
NS = 16
NW = NC * NS
EP = 331776
EPW = EP // NW
EPS = EP // NS
K = 96
NCHUNK = EPS // K
NB = 3
ZR = 80
RPT = NP // NS

_mesh = plsc.VectorSubcoreMesh(core_axis_name="c", subcore_axis_name="s")

_GDN = lax.GatherDimensionNumbers(
    offset_dims=(), collapsed_slice_dims=(0,), start_index_map=(0,))


def _bcast_lane(v16, lane):
    idx = jnp.full((16, 1), lane, dtype=jnp.int32)
    return lax.gather(v16, idx, _GDN, (1,),
                      mode=lax.GatherScatterMode.PROMISE_IN_BOUNDS)

_sc_params = pltpu.CompilerParams(needs_layout_passes=False,
                                  use_tc_tiling_on_sc=False)


@functools.partial(
    pl.kernel,
    out_type=jax.ShapeDtypeStruct((NW, NP), jnp.float32),
    mesh=_mesh,
    scratch_types=[
        pltpu.VMEM((EPW,), jnp.int32),
        pltpu.VMEM((EPW,), jnp.float32),
        pltpu.VMEM((NP,), jnp.float32),
    ],
    compiler_params=_sc_params,
)
def _deg(col_hbm, ew_hbm, degp_hbm, colv, ewv, degv):
    cid = lax.axis_index("c")
    sid = lax.axis_index("s")
    wid = sid * NC + cid
    zeros16 = jnp.zeros((16,), jnp.float32)

    @pl.loop(0, NP, step=16)
    def _(i):
        degv[pl.ds(i, 16)] = zeros16

    pltpu.sync_copy(col_hbm.at[wid], colv)
    pltpu.sync_copy(ew_hbm.at[wid], ewv)

    @pl.loop(0, EPW, step=16)
    def _(e):
        sl = pl.ds(e, 16)
        plsc.addupdate_scatter(degv, [colv[sl]], ewv[sl])

    pltpu.sync_copy(degv, degp_hbm.at[wid])


def _dis_body(p_ref, o_ref):
    s = jnp.sum(p_ref[...], axis=0, keepdims=True)
    o_ref[...] = jnp.where(s > 0, lax.rsqrt(jnp.maximum(s, 1e-30)), 0.0)


_dis = pl.pallas_call(
    _dis_body, out_shape=jax.ShapeDtypeStruct((1, NP), jnp.float32))


@functools.partial(
    pl.kernel,
    out_type=jax.ShapeDtypeStruct((NC, NP, DH), jnp.float32),
    mesh=_mesh,
    scratch_types=[
        pltpu.VMEM((NCHUNK, K), jnp.int32),
        pltpu.VMEM((NCHUNK, K), jnp.int32),
        pltpu.VMEM((NCHUNK, K), jnp.float32),
        pltpu.VMEM((NB, K, DH), jnp.float32),
        pltpu.VMEM_SHARED((NP, DH), jnp.float32),
        pltpu.SemaphoreType.DMA((NB,)),
        pltpu.SemaphoreType.DMA((NB,)),
    ],
    compiler_params=_sc_params,
)
def _agg(row_hbm, col_hbm, ew_hbm, table_hbm, out_hbm,
         rowv, colv, wv, msgv, acc, gsem, ssem):
    cid = lax.axis_index("c")
    sid = lax.axis_index("s")
    zeros16 = jnp.zeros((16,), jnp.float32)

    @pl.loop(0, K)
    def _(r):
        for q in range(DH // 16):
            msgv[0, r, pl.ds(q * 16, 16)] = zeros16

    for j in range(RPT // ZR):
        pltpu.sync_copy(msgv.at[0, pl.ds(0, ZR)],
                        acc.at[pl.ds(sid * RPT + j * ZR, ZR)])

    pltpu.sync_copy(row_hbm.at[sid], rowv)
    pltpu.sync_copy(col_hbm.at[sid], colv)
    pltpu.sync_copy(ew_hbm.at[sid], wv)

    table = table_hbm.at[cid]

    def _gather_start(c, b):
        pltpu.async_copy(table.at[rowv.at[c]], msgv.at[b], gsem.at[b])

    def _gather_wait(c, b):
        pltpu.make_async_copy(table.at[rowv.at[c]], msgv.at[b],
                              gsem.at[b]).wait()

    def _scatter_start(c, b):
        pltpu.async_copy(msgv.at[b], acc.at[colv.at[c]], ssem.at[b],
                         add=True)

    def _scatter_wait(c, b):
        pltpu.make_async_copy(msgv.at[b], acc.at[colv.at[c]],
                              ssem.at[b]).wait()

    _gather_start(0, 0)
    _gather_start(1, 1)
    plsc.subcore_barrier()

    @pl.loop(0, NCHUNK, step=NB)
    def _(c0):
        for b in range(NB):
            c = c0 + b
            _gather_wait(c, b)
            cvec = jnp.full((16,), c, dtype=jnp.int32)

            @plsc.parallel_loop(0, K, unroll=4)
            def _(j):
                jvec = jnp.full((16,), j, dtype=jnp.int32)
                wj = plsc.load_gather(wv, [cvec, jvec])
                for q in range(DH // 16):
                    sl = pl.ds(q * 16, 16)
                    msgv[b, j, sl] = msgv[b, j, sl] * wj

            _scatter_start(c, b)
            bn = (b + 2) % NB

            @pl.when(jnp.logical_and(c >= 1, c + 2 < NCHUNK))
            def _():
                _scatter_wait(c - 1, bn)

            @pl.when(c + 2 < NCHUNK)
            def _():
                _gather_start(c + 2, bn)

    for b in range(NB):
        _scatter_wait(NCHUNK - NB + b, b)
    plsc.subcore_barrier()
    pltpu.sync_copy(acc.at[pl.ds(sid * RPT, RPT)],
                    out_hbm.at[cid, pl.ds(sid * RPT, RPT)])


def _mm_body(x_ref, w_ref, d_ref, o_ref):
    r = jnp.dot(x_ref[...], w_ref[...], preferred_element_type=jnp.float32)
    r = r * d_ref[...]
    o_ref[0] = r[:, :DH]
    o_ref[1] = r[:, DH:]


_mm = pl.pallas_call(
    _mm_body, out_shape=jax.ShapeDtypeStruct((NC, NP, DH), jnp.float32))


def _relu_mm_body(a_ref, b_ref, w_ref, d_ref, o_ref):
    a = jnp.concatenate([a_ref[0], a_ref[1]], axis=1)
    h = jnp.maximum(a * d_ref[...] + b_ref[...], 0.0)
    r = jnp.dot(h, w_ref[...], preferred_element_type=jnp.float32)
    r = r * d_ref[...]
    o_ref[0] = r[:, :DH]
    o_ref[1] = r[:, DH:]


_relu_mm = pl.pallas_call(
    _relu_mm_body, out_shape=jax.ShapeDtypeStruct((NC, NP, DH), jnp.float32))


def _relu_body(a_ref, b_ref, d_ref, o_ref):
    a = jnp.concatenate([a_ref[0], a_ref[1]], axis=1)
    o_ref[...] = jnp.maximum(a * d_ref[...] + b_ref[...], 0.0)


_relu = pl.pallas_call(
    _relu_body, out_shape=jax.ShapeDtypeStruct((NP, D), jnp.float32))


def kernel(x, edge_index, edge_weight, W1, b1, W2, b2):
    row = edge_index[0].astype(jnp.int32)
    col = edge_index[1].astype(jnp.int32)
    loop = jnp.arange(N, dtype=jnp.int32)
    padi = jnp.full((EP - E - N,), NP - 1, dtype=jnp.int32)
    row_f = jnp.concatenate([row, loop, padi])
    col_f = jnp.concatenate([col, loop, padi])
    ew_f = jnp.concatenate([
        edge_weight.astype(jnp.float32),
        jnp.ones((N,), jnp.float32),
        jnp.zeros((EP - E - N,), jnp.float32),
    ])
    row_c = row_f.reshape(NS, NCHUNK, K)
    col_c = col_f.reshape(NS, NCHUNK, K)
    ew_c = ew_f.reshape(NS, NCHUNK, K)
    x_pad = jnp.concatenate([x, jnp.zeros((NP - N, D), x.dtype)])

    dis = _dis(_deg(col_f.reshape(NW, EPW), ew_f.reshape(NW, EPW)))
    dis_col = dis.reshape(NP, 1)
    y1 = _mm(x_pad, W1, dis_col)
    agg1 = _agg(row_c, col_c, ew_c, y1)
    y2 = _relu_mm(agg1, b1.reshape(1, D), W2, dis_col)
    agg2 = _agg(row_c, col_c, ew_c, y2)
    out = _relu(agg2, b2.reshape(1, D), dis_col)
    return out[:N]


# speedup vs baseline: 1.0007x; 1.0007x over previous
import dataclasses
import functools

import jax
import jax.numpy as jnp
import numpy as np
from jax import lax
from jax.experimental import pallas as pl
from jax.experimental.pallas import tpu as pltpu
from jax.experimental.pallas import tpu_sc as plsc

N = 10000
NP = 10240
E = 320000
D = 128
DH = 64
NC = 2
NS = 16
NW = NC * NS
EP = 331776
EPW = EP // NW
EPS = EP // NS
K = 96
NCHUNK = EPS // K
NB = 3
ZR = 80
RPT = NP // NS

_mesh = plsc.VectorSubcoreMesh(core_axis_name="c", subcore_axis_name="s")

_GDN = lax.GatherDimensionNumbers(
    offset_dims=(), collapsed_slice_dims=(0,), start_index_map=(0,))


def _bcast_lane(v16, lane):
    idx = jnp.full((16, 1), lane, dtype=jnp.int32)
    return lax.gather(v16, idx, _GDN, (1,),
                      mode=lax.GatherScatterMode.PROMISE_IN_BOUNDS)

_sc_params = pltpu.CompilerParams(needs_layout_passes=False,
                                  use_tc_tiling_on_sc=False)


@functools.partial(
    pl.kernel,
    out_type=jax.ShapeDtypeStruct((NW, NP), jnp.float32),
    mesh=_mesh,
    scratch_types=[
        pltpu.VMEM((EPW,), jnp.int32),
        pltpu.VMEM((EPW,), jnp.float32),
        pltpu.VMEM((NP,), jnp.float32),
    ],
    compiler_params=_sc_params,
)
def _deg(col_hbm, ew_hbm, degp_hbm, colv, ewv, degv):
    cid = lax.axis_index("c")
    sid = lax.axis_index("s")
    wid = sid * NC + cid
    zeros16 = jnp.zeros((16,), jnp.float32)

    @pl.loop(0, NP, step=16)
    def _(i):
        degv[pl.ds(i, 16)] = zeros16

    pltpu.sync_copy(col_hbm.at[wid], colv)
    pltpu.sync_copy(ew_hbm.at[wid], ewv)

    @pl.loop(0, EPW, step=16)
    def _(e):
        sl = pl.ds(e, 16)
        plsc.addupdate_scatter(degv, [colv[sl]], ewv[sl])

    pltpu.sync_copy(degv, degp_hbm.at[wid])


def _dis_body(p_ref, o_ref):
    s = jnp.sum(p_ref[...], axis=0, keepdims=True)
    o_ref[...] = jnp.where(s > 0, lax.rsqrt(jnp.maximum(s, 1e-30)), 0.0)


_dis = pl.pallas_call(
    _dis_body, out_shape=jax.ShapeDtypeStruct((1, NP), jnp.float32))


@functools.partial(
    pl.kernel,
    out_type=jax.ShapeDtypeStruct((NC, NP, DH), jnp.float32),
    mesh=_mesh,
    scratch_types=[
        pltpu.VMEM((EPS,), jnp.int32),
        pltpu.VMEM((EPS,), jnp.int32),
        pltpu.VMEM((EPS,), jnp.float32),
        pltpu.VMEM((NB, K, DH), jnp.float32),
        pltpu.VMEM_SHARED((NP, DH), jnp.float32),
        pltpu.SemaphoreType.DMA((NB,)),
        pltpu.SemaphoreType.DMA((NB,)),
    ],
    compiler_params=_sc_params,
)
def _agg(row_hbm, col_hbm, ew_hbm, table_hbm, out_hbm,
         rowv, colv, wv, msgv, acc, gsem, ssem):
    cid = lax.axis_index("c")
    sid = lax.axis_index("s")
    zeros16 = jnp.zeros((16,), jnp.float32)

    @pl.loop(0, K)
    def _(r):
        for q in range(DH // 16):
            msgv[0, r, pl.ds(q * 16, 16)] = zeros16

    for j in range(RPT // ZR):
        pltpu.sync_copy(msgv.at[0, pl.ds(0, ZR)],
                        acc.at[pl.ds(sid * RPT + j * ZR, ZR)])

    ebase = sid * EPS
    pltpu.sync_copy(row_hbm.at[pl.ds(ebase, EPS)], rowv)
    pltpu.sync_copy(col_hbm.at[pl.ds(ebase, EPS)], colv)
    pltpu.sync_copy(ew_hbm.at[pl.ds(ebase, EPS)], wv)

    table = table_hbm.at[cid]

    def _gather_start(c, b):
        pltpu.async_copy(table.at[rowv.at[pl.ds(c * K, K)]], msgv.at[b],
                         gsem.at[b])

    def _gather_wait(c, b):
        pltpu.make_async_copy(table.at[rowv.at[pl.ds(c * K, K)]], msgv.at[b],
                              gsem.at[b]).wait()

    def _scatter_start(c, b):
        pltpu.async_copy(msgv.at[b], acc.at[colv.at[pl.ds(c * K, K)]],
                         ssem.at[b], add=True)

    def _scatter_wait(c, b):
        pltpu.make_async_copy(msgv.at[b], acc.at[colv.at[pl.ds(c * K, K)]],
                              ssem.at[b]).wait()

    _gather_start(0, 0)
    _gather_start(1, 1)
    plsc.subcore_barrier()

    @pl.loop(0, NCHUNK, step=NB)
    def _(c0):
        for b in range(NB):
            c = c0 + b
            _gather_wait(c, b)
            cvec = jnp.full((16,), c * K, dtype=jnp.int32)

            @plsc.parallel_loop(0, K, unroll=4)
            def _(j):
                jvec = jnp.full((16,), j, dtype=jnp.int32)
                wj = plsc.load_gather(wv, [cvec + jvec])
                for q in range(DH // 16):
                    sl = pl.ds(q * 16, 16)
                    msgv[b, j, sl] = msgv[b, j, sl] * wj

            _scatter_start(c, b)
            bn = (b + 2) % NB

            @pl.when(jnp.logical_and(c >= 1, c + 2 < NCHUNK))
            def _():
                _scatter_wait(c - 1, bn)

            @pl.when(c + 2 < NCHUNK)
            def _():
                _gather_start(c + 2, bn)

    for b in range(NB):
        _scatter_wait(NCHUNK - NB + b, b)
    plsc.subcore_barrier()
    pltpu.sync_copy(acc.at[pl.ds(sid * RPT, RPT)],
                    out_hbm.at[cid, pl.ds(sid * RPT, RPT)])


def _mm_body(x_ref, w_ref, d_ref, o_ref):
    r = jnp.dot(x_ref[...], w_ref[...], preferred_element_type=jnp.float32)
    r = jnp.concatenate([r, jnp.zeros((NP - N, D), jnp.float32)])
    r = r * d_ref[...]
    o_ref[0] = r[:, :DH]
    o_ref[1] = r[:, DH:]


_mm = pl.pallas_call(
    _mm_body, out_shape=jax.ShapeDtypeStruct((NC, NP, DH), jnp.float32))


def _relu_mm_body(a_ref, b_ref, w_ref, d_ref, o_ref):
    a = jnp.concatenate([a_ref[0], a_ref[1]], axis=1)
    h = jnp.maximum(a * d_ref[...] + b_ref[...], 0.0)
    r = jnp.dot(h, w_ref[...], preferred_element_type=jnp.float32)
    r = r * d_ref[...]
    o_ref[0] = r[:, :DH]
    o_ref[1] = r[:, DH:]


_relu_mm = pl.pallas_call(
    _relu_mm_body, out_shape=jax.ShapeDtypeStruct((NC, NP, DH), jnp.float32))


def _relu_body(a_ref, b_ref, d_ref, o_ref):
    a = jnp.concatenate([a_ref[0], a_ref[1]], axis=1)
    o_ref[...] = jnp.maximum(a * d_ref[...] + b_ref[...], 0.0)


_relu = pl.pallas_call(
    _relu_body, out_shape=jax.ShapeDtypeStruct((NP, D), jnp.float32))


def kernel(x, edge_index, edge_weight, W1, b1, W2, b2):
    row = edge_index[0].astype(jnp.int32)
    col = edge_index[1].astype(jnp.int32)
    loop = jnp.arange(N, dtype=jnp.int32)
    padi = jnp.full((EP - E - N,), NP - 1, dtype=jnp.int32)
    row_f = jnp.concatenate([row, loop, padi])
    col_f = jnp.concatenate([col, loop, padi])
    ew_f = jnp.concatenate([
        edge_weight.astype(jnp.float32),
        jnp.ones((N,), jnp.float32),
        jnp.zeros((EP - E - N,), jnp.float32),
    ])
    dis = _dis(_deg(col_f.reshape(NW, EPW), ew_f.reshape(NW, EPW)))
    dis_col = dis.reshape(NP, 1)
    y1 = _mm(x, W1, dis_col)
    agg1 = _agg(row_f, col_f, ew_f, y1)
    y2 = _relu_mm(agg1, b1.reshape(1, D), W2, dis_col)
    agg2 = _agg(row_f, col_f, ew_f, y2)
    out = _relu(agg2, b2.reshape(1, D), dis_col)
    return out[:N]


# speedup vs baseline: 1.1149x; 1.1141x over previous
import dataclasses
import functools

import jax
import jax.numpy as jnp
import numpy as np
from jax import lax
from jax.experimental import pallas as pl
from jax.experimental.pallas import tpu as pltpu
from jax.experimental.pallas import tpu_sc as plsc

N = 10000
NP = 10240
E = 320000
D = 128
DH = 64
NC = 2
NS = 16
NW = NC * NS
EP = 330240
EPW = EP // NW
EPS = EP // NS
K = 80
NCHUNK = EPS // K
NB = 3
ZR = 80
RPT = NP // NS

_mesh = plsc.VectorSubcoreMesh(core_axis_name="c", subcore_axis_name="s")

_GDN = lax.GatherDimensionNumbers(
    offset_dims=(), collapsed_slice_dims=(0,), start_index_map=(0,))


def _bcast_lane(v16, lane):
    idx = jnp.full((16, 1), lane, dtype=jnp.int32)
    return lax.gather(v16, idx, _GDN, (1,),
                      mode=lax.GatherScatterMode.PROMISE_IN_BOUNDS)

_sc_params = pltpu.CompilerParams(needs_layout_passes=False,
                                  use_tc_tiling_on_sc=False)


@functools.partial(
    pl.kernel,
    out_type=jax.ShapeDtypeStruct((NW, NP), jnp.float32),
    mesh=_mesh,
    scratch_types=[
        pltpu.VMEM((EPW,), jnp.int32),
        pltpu.VMEM((EPW,), jnp.float32),
        pltpu.VMEM((NP,), jnp.float32),
    ],
    compiler_params=_sc_params,
)
def _deg(col_hbm, ew_hbm, degp_hbm, colv, ewv, degv):
    cid = lax.axis_index("c")
    sid = lax.axis_index("s")
    wid = sid * NC + cid
    zeros16 = jnp.zeros((16,), jnp.float32)

    @pl.loop(0, NP, step=16)
    def _(i):
        degv[pl.ds(i, 16)] = zeros16

    pltpu.sync_copy(col_hbm.at[wid], colv)
    pltpu.sync_copy(ew_hbm.at[wid], ewv)

    @pl.loop(0, EPW, step=16)
    def _(e):
        sl = pl.ds(e, 16)
        plsc.addupdate_scatter(degv, [colv[sl]], ewv[sl])

    pltpu.sync_copy(degv, degp_hbm.at[wid])


def _dis_body(p_ref, o_ref):
    s = jnp.sum(p_ref[...], axis=0, keepdims=True)
    o_ref[...] = jnp.where(s > 0, lax.rsqrt(jnp.maximum(s, 1e-30)), 0.0)


_dis = pl.pallas_call(
    _dis_body, out_shape=jax.ShapeDtypeStruct((1, NP), jnp.float32))


@functools.partial(
    pl.kernel,
    out_type=jax.ShapeDtypeStruct((NC, NP, DH), jnp.float32),
    mesh=_mesh,
    scratch_types=[
        pltpu.VMEM((EPS,), jnp.int32),
        pltpu.VMEM((EPS,), jnp.int32),
        pltpu.VMEM((EPS,), jnp.float32),
        pltpu.VMEM((NP,), jnp.float32),
        pltpu.VMEM((NB, K, DH), jnp.float32),
        pltpu.VMEM_SHARED((NP, DH), jnp.float32),
        pltpu.SemaphoreType.DMA((NB,)),
        pltpu.SemaphoreType.DMA((NB,)),
    ],
    compiler_params=_sc_params,
)
def _agg(row_hbm, col_hbm, ew_hbm, dis_hbm, table_hbm, out_hbm,
         rowv, colv, wv, disv, msgv, acc, gsem, ssem):
    cid = lax.axis_index("c")
    sid = lax.axis_index("s")
    zeros16 = jnp.zeros((16,), jnp.float32)

    @pl.loop(0, K)
    def _(r):
        for q in range(DH // 16):
            msgv[0, r, pl.ds(q * 16, 16)] = zeros16

    for j in range(RPT // ZR):
        pltpu.sync_copy(msgv.at[0, pl.ds(0, ZR)],
                        acc.at[pl.ds(sid * RPT + j * ZR, ZR)])

    pltpu.sync_copy(dis_hbm.at[0], disv)
    ebase = sid * EPS
    pltpu.sync_copy(row_hbm.at[pl.ds(ebase, EPS)], rowv)
    pltpu.sync_copy(col_hbm.at[pl.ds(ebase, EPS)], colv)
    pltpu.sync_copy(ew_hbm.at[pl.ds(ebase, EPS)], wv)

    @plsc.parallel_loop(0, EPS, step=16, unroll=2)
    def _(e):
        sl = pl.ds(e, 16)
        dr = plsc.load_gather(disv, [rowv[sl]])
        dc = plsc.load_gather(disv, [colv[sl]])
        wv[sl] = wv[sl] * dr * dc

    table = table_hbm.at[cid]

    def _gather_start(c, b):
        pltpu.async_copy(table.at[rowv.at[pl.ds(c * K, K)]], msgv.at[b],
                         gsem.at[b])

    def _gather_wait(c, b):
        pltpu.make_async_copy(table.at[rowv.at[pl.ds(c * K, K)]], msgv.at[b],
                              gsem.at[b]).wait()

    def _scatter_start(c, b):
        pltpu.async_copy(msgv.at[b], acc.at[colv.at[pl.ds(c * K, K)]],
                         ssem.at[b], add=True)

    def _scatter_wait(c, b):
        pltpu.make_async_copy(msgv.at[b], acc.at[colv.at[pl.ds(c * K, K)]],
                              ssem.at[b]).wait()

    _gather_start(0, 0)
    _gather_start(1, 1)
    plsc.subcore_barrier()

    @pl.loop(0, NCHUNK, step=NB)
    def _(c0):
        for b in range(NB):
            c = c0 + b
            _gather_wait(c, b)
            cvec = jnp.full((16,), c * K, dtype=jnp.int32)

            @plsc.parallel_loop(0, K, unroll=4)
            def _(j):
                jvec = jnp.full((16,), j, dtype=jnp.int32)
                wj = plsc.load_gather(wv, [cvec + jvec])
                for q in range(DH // 16):
                    sl = pl.ds(q * 16, 16)
                    msgv[b, j, sl] = msgv[b, j, sl] * wj

            _scatter_start(c, b)
            bn = (b + 2) % NB

            @pl.when(jnp.logical_and(c >= 1, c + 2 < NCHUNK))
            def _():
                _scatter_wait(c - 1, bn)

            @pl.when(c + 2 < NCHUNK)
            def _():
                _gather_start(c + 2, bn)

    for b in range(NB):
        _scatter_wait(NCHUNK - NB + b, b)
    plsc.subcore_barrier()
    pltpu.sync_copy(acc.at[pl.ds(sid * RPT, RPT)],
                    out_hbm.at[cid, pl.ds(sid * RPT, RPT)])


def _mm_body(x_ref, w_ref, o_ref):
    r = jnp.dot(x_ref[...], w_ref[...], preferred_element_type=jnp.float32)
    r = jnp.concatenate([r, jnp.zeros((NP - N, D), jnp.float32)])
    o_ref[0] = r[:, :DH]
    o_ref[1] = r[:, DH:]


_mm = pl.pallas_call(
    _mm_body, out_shape=jax.ShapeDtypeStruct((NC, NP, DH), jnp.float32))


def _relu_mm_body(a_ref, b_ref, w_ref, o_ref):
    a = jnp.concatenate([a_ref[0], a_ref[1]], axis=1)
    h = jnp.maximum(a + b_ref[...], 0.0)
    r = jnp.dot(h, w_ref[...], preferred_element_type=jnp.float32)
    o_ref[0] = r[:, :DH]
    o_ref[1] = r[:, DH:]


_relu_mm = pl.pallas_call(
    _relu_mm_body, out_shape=jax.ShapeDtypeStruct((NC, NP, DH), jnp.float32))


def _relu_body(a_ref, b_ref, o_ref):
    a = jnp.concatenate([a_ref[0], a_ref[1]], axis=1)
    o_ref[...] = jnp.maximum(a + b_ref[...], 0.0)


_relu = pl.pallas_call(
    _relu_body, out_shape=jax.ShapeDtypeStruct((NP, D), jnp.float32))


def kernel(x, edge_index, edge_weight, W1, b1, W2, b2):
    row = edge_index[0].astype(jnp.int32)
    col = edge_index[1].astype(jnp.int32)
    loop = jnp.arange(N, dtype=jnp.int32)
    padi = jnp.full((EP - E - N,), NP - 1, dtype=jnp.int32)
    row_f = jnp.concatenate([row, loop, padi])
    col_f = jnp.concatenate([col, loop, padi])
    ew_f = jnp.concatenate([
        edge_weight.astype(jnp.float32),
        jnp.ones((N,), jnp.float32),
        jnp.zeros((EP - E - N,), jnp.float32),
    ])
    dis = _dis(_deg(col_f.reshape(NW, EPW), ew_f.reshape(NW, EPW)))
    y1 = _mm(x, W1)
    agg1 = _agg(row_f, col_f, ew_f, dis, y1)
    y2 = _relu_mm(agg1, b1.reshape(1, D), W2)
    agg2 = _agg(row_f, col_f, ew_f, dis, y2)
    out = _relu(agg2, b2.reshape(1, D))
    return out[:N]
